# Initial kernel scaffold; baseline (speedup 1.0000x reference)
import jax
import jax.numpy as jnp
from jax.experimental import pallas as pl


def kernel(x, pe):
    raise NotImplementedError("write your pallas kernel here")


# speedup vs baseline: 4.7473x; 4.7473x over previous
import jax
import jax.numpy as jnp
from jax.experimental import pallas as pl

_ROW_BLOCK = 256


def _bcast_copy(pe_ref, o_ref):
    o_ref[...] = jnp.broadcast_to(pe_ref[...][None, :, :], o_ref.shape)


def kernel(x, pe):
    batch, seq_len = x.shape
    embed = pe.shape[1]
    rb = _ROW_BLOCK if seq_len % _ROW_BLOCK == 0 else seq_len
    return pl.pallas_call(
        _bcast_copy,
        grid=(seq_len // rb,),
        in_specs=[pl.BlockSpec((rb, embed), lambda i: (i, 0))],
        out_specs=pl.BlockSpec((batch, rb, embed), lambda i: (0, i, 0)),
        out_shape=jax.ShapeDtypeStruct((batch, seq_len, embed), pe.dtype),
    )(pe[:seq_len])


# speedup vs baseline: 5.0350x; 1.0606x over previous
import jax
import jax.numpy as jnp
from jax.experimental import pallas as pl

_ROW_BLOCK = 512


def _bcast_copy(pe_ref, o_ref):
    o_ref[...] = jnp.broadcast_to(pe_ref[...][None, :, :], o_ref.shape)


def kernel(x, pe):
    batch, seq_len = x.shape
    embed = pe.shape[1]
    rb = _ROW_BLOCK if seq_len % _ROW_BLOCK == 0 else seq_len
    return pl.pallas_call(
        _bcast_copy,
        grid=(seq_len // rb,),
        in_specs=[pl.BlockSpec((rb, embed), lambda i: (i, 0))],
        out_specs=pl.BlockSpec((batch, rb, embed), lambda i: (0, i, 0)),
        out_shape=jax.ShapeDtypeStruct((batch, seq_len, embed), pe.dtype),
    )(pe[:seq_len])
